# baseline (speedup 1.0000x reference)
import functools

import jax
import jax.numpy as jnp
from jax import lax
from jax.experimental import pallas as pl
from jax.experimental.pallas import tpu as pltpu
from jax.experimental.pallas import tpu_sc as plsc

NC = 2
NS = 16
NW = NC * NS
BS = 128
TN = 1000


def _sc_gather(table, idx3, nblk_per_w, h):
    assert nblk_per_w % 2 == 1
    mesh = plsc.VectorSubcoreMesh(core_axis_name="c", subcore_axis_name="s")
    total = NW * nblk_per_w * BS

    @functools.partial(
        pl.kernel,
        out_type=jax.ShapeDtypeStruct((total, h), jnp.float32),
        mesh=mesh,
        scratch_types=[
            pltpu.VMEM((nblk_per_w, BS), jnp.int32),
            pltpu.VMEM((BS, h), jnp.float32),
            pltpu.VMEM((BS, h), jnp.float32),
            pltpu.SemaphoreType.DMA,
            pltpu.SemaphoreType.DMA,
        ],
    )
    def k(table_hbm, idx_hbm, out_hbm, idx_v, buf0, buf1, sem0, sem1):
        wid = lax.axis_index("s") * NC + lax.axis_index("c")
        pltpu.sync_copy(idx_hbm.at[wid], idx_v)

        def start(j, buf, sem):
            pltpu.async_copy(table_hbm.at[idx_v.at[j]], buf, sem)

        def wait_store(j, buf, sem):
            pltpu.make_async_copy(table_hbm.at[idx_v.at[0]], buf, sem).wait()
            pltpu.sync_copy(buf, out_hbm.at[pl.ds((j * NW + wid) * BS, BS)])

        start(0, buf0, sem0)

        @pl.loop(0, nblk_per_w - 1, step=2)
        def _(j0):
            start(j0 + 1, buf1, sem1)
            wait_store(j0, buf0, sem0)
            start(j0 + 2, buf0, sem0)
            wait_store(j0 + 1, buf1, sem1)

        wait_store(nblk_per_w - 1, buf0, sem0)

    return k(table, idx3)


def _mlp_body(x_ref, g_ref, d_ref, w1a_ref, w1b_ref, b1_ref, w2_ref, b2_ref,
              o_ref):
    inv = 1.0 / (d_ref[...] + 1e-8)
    h = jnp.dot(x_ref[0], w1a_ref[...], preferred_element_type=jnp.float32)
    h = h + jnp.dot(g_ref[...] * inv, w1b_ref[...],
                    preferred_element_type=jnp.float32)
    h = jnp.maximum(h + b1_ref[...], 0.0)
    o_ref[0] = (jnp.dot(h, w2_ref[...], preferred_element_type=jnp.float32)
                + b2_ref[...])


def _mlp_body_alias(x_ref, g_ref, d_ref, w1a_ref, w1b_ref, b1_ref, w2_ref,
                    b2_ref, _obuf_ref, o_ref):
    _mlp_body(x_ref, g_ref, d_ref, w1a_ref, w1b_ref, b1_ref, w2_ref, b2_ref,
              o_ref)


def kernel(x, x_scale, fine2coarse_index, distances, W1, b1, W2, b2):
    B, N, H = x.shape
    R = B * N
    NCHUNK = 2
    S = R // NCHUNK
    cpb = NCHUNK // B
    assert S % TN == 0 and N % S == 0
    nblk_c = -(-S // BS)
    nblk_c = -(-nblk_c // NW) * NW
    nblk_w = nblk_c // NW
    Spad = nblk_c * BS

    idx = fine2coarse_index.astype(jnp.int32)
    table = x_scale.reshape(R, H)
    w1a, w1b = W1[:H], W1[H:]
    b1r, b2r = b1.reshape(1, H), b2.reshape(1, H)
    dist = distances.reshape(N, 1)
    nb_c = S // TN

    gathered = []
    for c in range(NCHUNK):
        b = (c * S) // N
        o = (c * S) % N
        idx_c = lax.dynamic_slice(idx, (o,), (S,)) + b * N
        idx_pad = jnp.concatenate([idx_c, jnp.zeros(Spad - S, jnp.int32)])
        idx3 = idx_pad.reshape(nblk_w, NW, BS).transpose(1, 0, 2)
        gathered.append(_sc_gather(table, idx3, nblk_w, H))

    out = None
    for c in range(NCHUNK):
        b = c // cpb
        h0 = c % cpb
        x_map = functools.partial(lambda b, h0, n: (b, h0 * nb_c + n, 0), b, h0)
        d_map = functools.partial(lambda h0, n: (h0 * nb_c + n, 0), h0)
        in_specs = [
            pl.BlockSpec((1, TN, H), x_map),
            pl.BlockSpec((TN, H), lambda n: (n, 0)),
            pl.BlockSpec((TN, 1), d_map),
            pl.BlockSpec((H, H), lambda n: (0, 0)),
            pl.BlockSpec((H, H), lambda n: (0, 0)),
            pl.BlockSpec((1, H), lambda n: (0, 0)),
            pl.BlockSpec((H, H), lambda n: (0, 0)),
            pl.BlockSpec((1, H), lambda n: (0, 0)),
        ]
        args = [x, gathered[c], dist, w1a, w1b, b1r, W2, b2r]
        body = _mlp_body
        alias = {}
        if out is not None:
            in_specs.append(pl.BlockSpec(memory_space=pl.ANY))
            args.append(out)
            body = _mlp_body_alias
            alias = {8: 0}
        out = pl.pallas_call(
            body,
            grid=(nb_c,),
            in_specs=in_specs,
            out_specs=pl.BlockSpec((1, TN, H), x_map),
            out_shape=jax.ShapeDtypeStruct((B, N, H), jnp.float32),
            input_output_aliases=alias,
            compiler_params=pltpu.CompilerParams(
                dimension_semantics=("parallel",)),
        )(*args)
    return out


# speedup vs baseline: 1.4284x; 1.4284x over previous
import functools

import jax
import jax.numpy as jnp
from jax import lax
from jax.experimental import pallas as pl
from jax.experimental.pallas import tpu as pltpu
from jax.experimental.pallas import tpu_sc as plsc

NC = 2
NS = 16
NW = NC * NS
BS = 128
NBUF = 3
LA = NBUF - 1


def _sc_gather(table, idx3, nblk_per_w, h):
    nb = nblk_per_w
    assert nb >= 8
    mesh = plsc.VectorSubcoreMesh(core_axis_name="c", subcore_axis_name="s")
    total = NW * nb * BS

    @functools.partial(
        pl.kernel,
        out_type=jax.ShapeDtypeStruct((total, h), jnp.float32),
        mesh=mesh,
        scratch_types=[
            pltpu.VMEM((nb, BS), jnp.int32),
            [pltpu.VMEM((BS, h), jnp.float32)] * NBUF,
            [pltpu.SemaphoreType.DMA] * NBUF,
            [pltpu.SemaphoreType.DMA] * NBUF,
        ],
    )
    def k(table_hbm, idx_hbm, out_hbm, idx_v, bufs, gsems, ssems):
        wid = lax.axis_index("s") * NC + lax.axis_index("c")
        pltpu.sync_copy(idx_hbm.at[wid], idx_v)

        def start_gather(j, i):
            pltpu.async_copy(table_hbm.at[idx_v.at[j]], bufs[i], gsems[i])

        def wait_gather(i):
            pltpu.make_async_copy(
                table_hbm.at[idx_v.at[0]], bufs[i], gsems[i]).wait()

        def start_store(j, i):
            pltpu.async_copy(
                bufs[i], out_hbm.at[pl.ds((j * NW + wid) * BS, BS)], ssems[i])

        def wait_store(i):
            pltpu.make_async_copy(
                bufs[i], out_hbm.at[pl.ds(0, BS)], ssems[i]).wait()

        def step_full(j):
            i, i3 = j % NBUF, (j + LA) % NBUF
            wait_gather(i)
            start_store(j, i)
            wait_store(i3)
            start_gather(j + LA, i3)

        def step_nofetch(j):
            wait_gather(j % NBUF)
            start_store(j, j % NBUF)

        E = ((nb - LA) // NBUF) * NBUF
        for j in range(LA):
            start_gather(j, j)
        wait_gather(0)
        start_store(0, 0)
        start_gather(LA, LA)
        for j in range(1, NBUF):
            step_full(j)

        @pl.loop(NBUF, E, step=NBUF)
        def _(j0):
            for i in range(NBUF):
                wait_gather(i)
                start_store(j0 + i, i)
                wait_store((i + LA) % NBUF)
                start_gather(j0 + i + LA, (i + LA) % NBUF)

        for j in range(E, nb - LA):
            step_full(j)
        for j in range(nb - LA, nb):
            step_nofetch(j)
        for i in range(NBUF):
            wait_store(i)

    return k(table, idx3)


def _mlp_body(x_ref, g_ref, d_ref, w1a_ref, w1b_ref, b1_ref, w2_ref, b2_ref,
              o_ref):
    inv = 1.0 / (d_ref[...] + 1e-8)
    h = jnp.dot(x_ref[0], w1a_ref[...], preferred_element_type=jnp.float32)
    h = h + jnp.dot(g_ref[...] * inv, w1b_ref[...],
                    preferred_element_type=jnp.float32)
    h = jnp.maximum(h + b1_ref[...], 0.0)
    o_ref[0] = (jnp.dot(h, w2_ref[...], preferred_element_type=jnp.float32)
                + b2_ref[...])


def kernel(x, x_scale, fine2coarse_index, distances, W1, b1, W2, b2):
    B, N, H = x.shape
    R = B * N
    nblk = -(-R // BS)
    nblk = -(-nblk // NW) * NW
    Rpad = nblk * BS
    nblk_per_w = nblk // NW

    idx = fine2coarse_index.astype(jnp.int32)
    offs = (jnp.arange(B, dtype=jnp.int32) * N)[:, None]
    idx_all = (idx[None, :] + offs).reshape(R)
    idx_pad = jnp.concatenate([idx_all, jnp.zeros(Rpad - R, jnp.int32)])
    idx3 = idx_pad.reshape(nblk_per_w, NW, BS).transpose(1, 0, 2)
    table = x_scale.reshape(R, H)

    gathered = _sc_gather(table, idx3, nblk_per_w, H)

    TN = 1000
    nb = N // TN
    out = pl.pallas_call(
        _mlp_body,
        grid=(B, nb),
        in_specs=[
            pl.BlockSpec((1, TN, H), lambda b, n: (b, n, 0)),
            pl.BlockSpec((TN, H), lambda b, n: (b * nb + n, 0)),
            pl.BlockSpec((TN, 1), lambda b, n: (n, 0)),
            pl.BlockSpec((H, H), lambda b, n: (0, 0)),
            pl.BlockSpec((H, H), lambda b, n: (0, 0)),
            pl.BlockSpec((1, H), lambda b, n: (0, 0)),
            pl.BlockSpec((H, H), lambda b, n: (0, 0)),
            pl.BlockSpec((1, H), lambda b, n: (0, 0)),
        ],
        out_specs=pl.BlockSpec((1, TN, H), lambda b, n: (b, n, 0)),
        out_shape=jax.ShapeDtypeStruct((B, N, H), jnp.float32),
        compiler_params=pltpu.CompilerParams(
            dimension_semantics=("parallel", "parallel")),
    )(x, gathered, distances.reshape(N, 1), W1[:H], W1[H:],
      b1.reshape(1, H), W2, b2.reshape(1, H))
    return out


# speedup vs baseline: 1.5086x; 1.0562x over previous
import functools

import jax
import jax.numpy as jnp
from jax import lax
from jax.experimental import pallas as pl
from jax.experimental.pallas import tpu as pltpu
from jax.experimental.pallas import tpu_sc as plsc

NC = 2
NS = 16
NW = NC * NS
BS = 128
NBUF = 3
LA = NBUF - 1


def _sc_gather(table, idx3, nblk_per_w, h):
    nb = nblk_per_w
    assert nb >= 8
    mesh = plsc.VectorSubcoreMesh(core_axis_name="c", subcore_axis_name="s")
    total = NW * nb * BS

    @functools.partial(
        pl.kernel,
        out_type=jax.ShapeDtypeStruct((total, h), jnp.float32),
        mesh=mesh,
        scratch_types=[
            pltpu.VMEM((nb, BS), jnp.int32),
            [pltpu.VMEM((BS, h), jnp.float32)] * NBUF,
            [pltpu.SemaphoreType.DMA] * NBUF,
            [pltpu.SemaphoreType.DMA] * NBUF,
        ],
    )
    def k(table_hbm, idx_hbm, out_hbm, idx_v, bufs, gsems, ssems):
        wid = lax.axis_index("s") * NC + lax.axis_index("c")
        pltpu.sync_copy(idx_hbm.at[wid], idx_v)

        def start_gather(j, i):
            pltpu.async_copy(table_hbm.at[idx_v.at[j]], bufs[i], gsems[i])

        def wait_gather(i):
            pltpu.make_async_copy(
                table_hbm.at[idx_v.at[0]], bufs[i], gsems[i]).wait()

        def start_store(j, i):
            pltpu.async_copy(
                bufs[i], out_hbm.at[pl.ds((j * NW + wid) * BS, BS)], ssems[i])

        def wait_store(i):
            pltpu.make_async_copy(
                bufs[i], out_hbm.at[pl.ds(0, BS)], ssems[i]).wait()

        def step_full(j):
            i, i3 = j % NBUF, (j + LA) % NBUF
            wait_gather(i)
            start_store(j, i)
            wait_store(i3)
            start_gather(j + LA, i3)

        def step_nofetch(j):
            wait_gather(j % NBUF)
            start_store(j, j % NBUF)

        E = ((nb - LA) // NBUF) * NBUF
        for j in range(LA):
            start_gather(j, j)
        wait_gather(0)
        start_store(0, 0)
        start_gather(LA, LA)
        for j in range(1, NBUF):
            step_full(j)

        @pl.loop(NBUF, E, step=NBUF)
        def _(j0):
            for i in range(NBUF):
                wait_gather(i)
                start_store(j0 + i, i)
                wait_store((i + LA) % NBUF)
                start_gather(j0 + i + LA, (i + LA) % NBUF)

        for j in range(E, nb - LA):
            step_full(j)
        for j in range(nb - LA, nb):
            step_nofetch(j)
        for i in range(NBUF):
            wait_store(i)

    return k(table, idx3)


def _mlp_body(x_ref, g_ref, d_ref, w1a_ref, w1b_ref, b1_ref, w2_ref, b2_ref,
              o_ref):
    inv = (1.0 / (d_ref[...] + 1e-8)).reshape(-1, 1)
    h = jnp.dot(x_ref[0], w1a_ref[...], preferred_element_type=jnp.float32)
    h = h + inv * jnp.dot(g_ref[...], w1b_ref[...],
                          preferred_element_type=jnp.float32)
    h = jnp.maximum(h + b1_ref[...], 0.0)
    o_ref[0] = (jnp.dot(h, w2_ref[...], preferred_element_type=jnp.float32)
                + b2_ref[...])


def kernel(x, x_scale, fine2coarse_index, distances, W1, b1, W2, b2):
    B, N, H = x.shape
    R = B * N
    nblk = -(-R // BS)
    nblk = -(-nblk // NW) * NW
    Rpad = nblk * BS
    nblk_per_w = nblk // NW

    idx = fine2coarse_index.astype(jnp.int32)
    offs = (jnp.arange(B, dtype=jnp.int32) * N)[:, None]
    idx_all = (idx[None, :] + offs).reshape(R)
    idx_pad = jnp.concatenate([idx_all, jnp.zeros(Rpad - R, jnp.int32)])
    idx3 = idx_pad.reshape(nblk_per_w, NW, BS).transpose(1, 0, 2)
    table = x_scale.reshape(R, H)

    gathered = _sc_gather(table, idx3, nblk_per_w, H)

    TN = 1000
    nb = N // TN
    out = pl.pallas_call(
        _mlp_body,
        grid=(B, nb),
        in_specs=[
            pl.BlockSpec((1, TN, H), lambda b, n: (b, n, 0)),
            pl.BlockSpec((TN, H), lambda b, n: (b * nb + n, 0)),
            pl.BlockSpec((1, 1, TN), lambda b, n: (n, 0, 0)),
            pl.BlockSpec((H, H), lambda b, n: (0, 0)),
            pl.BlockSpec((H, H), lambda b, n: (0, 0)),
            pl.BlockSpec((1, H), lambda b, n: (0, 0)),
            pl.BlockSpec((H, H), lambda b, n: (0, 0)),
            pl.BlockSpec((1, H), lambda b, n: (0, 0)),
        ],
        out_specs=pl.BlockSpec((1, TN, H), lambda b, n: (b, n, 0)),
        out_shape=jax.ShapeDtypeStruct((B, N, H), jnp.float32),
        compiler_params=pltpu.CompilerParams(
            dimension_semantics=("parallel", "parallel")),
    )(x, gathered, distances.reshape(nb, 1, TN), W1[:H], W1[H:],
      b1.reshape(1, H), W2, b2.reshape(1, H))
    return out


# speedup vs baseline: 1.6732x; 1.1091x over previous
import functools

import jax
import jax.numpy as jnp
from jax import lax
from jax.experimental import pallas as pl
from jax.experimental.pallas import tpu as pltpu
from jax.experimental.pallas import tpu_sc as plsc

NC = 2
NS = 16
NW = NC * NS
BS = 128
NBUF = 3
LA = NBUF - 1


def _sc_gather(table, idx3, nblk_per_w, h):
    nb = nblk_per_w
    assert nb >= 8
    mesh = plsc.VectorSubcoreMesh(core_axis_name="c", subcore_axis_name="s")
    total = NW * nb * BS

    @functools.partial(
        pl.kernel,
        out_type=jax.ShapeDtypeStruct((total, h), jnp.float32),
        mesh=mesh,
        scratch_types=[
            pltpu.VMEM((nb, BS), jnp.int32),
            [pltpu.VMEM((BS, h), jnp.float32)] * NBUF,
            [pltpu.SemaphoreType.DMA] * NBUF,
            [pltpu.SemaphoreType.DMA] * NBUF,
        ],
    )
    def k(table_hbm, idx_hbm, out_hbm, idx_v, bufs, gsems, ssems):
        wid = lax.axis_index("s") * NC + lax.axis_index("c")
        pltpu.sync_copy(idx_hbm.at[wid], idx_v)

        def start_gather(j, i):
            pltpu.async_copy(table_hbm.at[idx_v.at[j]], bufs[i], gsems[i])

        def wait_gather(i):
            pltpu.make_async_copy(
                table_hbm.at[idx_v.at[0]], bufs[i], gsems[i]).wait()

        def start_store(j, i):
            pltpu.async_copy(
                bufs[i], out_hbm.at[pl.ds((j * NW + wid) * BS, BS)], ssems[i])

        def wait_store(i):
            pltpu.make_async_copy(
                bufs[i], out_hbm.at[pl.ds(0, BS)], ssems[i]).wait()

        def step_full(j):
            i, i3 = j % NBUF, (j + LA) % NBUF
            wait_gather(i)
            start_store(j, i)
            wait_store(i3)
            start_gather(j + LA, i3)

        def step_nofetch(j):
            wait_gather(j % NBUF)
            start_store(j, j % NBUF)

        E = ((nb - LA) // NBUF) * NBUF
        for j in range(LA):
            start_gather(j, j)
        wait_gather(0)
        start_store(0, 0)
        start_gather(LA, LA)
        for j in range(1, NBUF):
            step_full(j)

        @pl.loop(NBUF, E, step=NBUF)
        def _(j0):
            for i in range(NBUF):
                wait_gather(i)
                start_store(j0 + i, i)
                wait_store((i + LA) % NBUF)
                start_gather(j0 + i + LA, (i + LA) % NBUF)

        for j in range(E, nb - LA):
            step_full(j)
        for j in range(nb - LA, nb):
            step_nofetch(j)
        for i in range(NBUF):
            wait_store(i)

    return k(table, idx3)


def _mlp_body(x_ref, g_ref, d_ref, w1a_ref, w1b_ref, b1_ref, w2_ref, b2_ref,
              o_ref):
    inv = (1.0 / (d_ref[...] + 1e-8)).reshape(-1, 1)
    h = jnp.dot(x_ref[0], w1a_ref[...], preferred_element_type=jnp.float32)
    h = h + inv * jnp.dot(g_ref[...], w1b_ref[...],
                          preferred_element_type=jnp.float32)
    h = jnp.maximum(h + b1_ref[...], 0.0)
    o_ref[0] = (jnp.dot(h, w2_ref[...], preferred_element_type=jnp.float32)
                + b2_ref[...])


def kernel(x, x_scale, fine2coarse_index, distances, W1, b1, W2, b2):
    B, N, H = x.shape
    R = B * N
    nblk = -(-R // BS)
    nblk = -(-nblk // NW) * NW
    Rpad = nblk * BS
    nblk_per_w = nblk // NW

    idx = fine2coarse_index.astype(jnp.int32)
    offs = (jnp.arange(B, dtype=jnp.int32) * N)[:, None]
    idx_all = (idx[None, :] + offs).reshape(R)
    idx_pad = jnp.concatenate([idx_all, jnp.zeros(Rpad - R, jnp.int32)])
    idx3 = idx_pad.reshape(nblk_per_w, NW, BS).transpose(1, 0, 2)
    table = x_scale.reshape(R, H)

    gathered = _sc_gather(table, idx3, nblk_per_w, H)

    TN = 2000
    nb = N // TN
    out = pl.pallas_call(
        _mlp_body,
        grid=(B, nb),
        in_specs=[
            pl.BlockSpec((1, TN, H), lambda b, n: (b, n, 0)),
            pl.BlockSpec((TN, H), lambda b, n: (b * nb + n, 0)),
            pl.BlockSpec((1, 1, TN), lambda b, n: (n, 0, 0)),
            pl.BlockSpec((H, H), lambda b, n: (0, 0)),
            pl.BlockSpec((H, H), lambda b, n: (0, 0)),
            pl.BlockSpec((1, H), lambda b, n: (0, 0)),
            pl.BlockSpec((H, H), lambda b, n: (0, 0)),
            pl.BlockSpec((1, H), lambda b, n: (0, 0)),
        ],
        out_specs=pl.BlockSpec((1, TN, H), lambda b, n: (b, n, 0)),
        out_shape=jax.ShapeDtypeStruct((B, N, H), jnp.float32),
        compiler_params=pltpu.CompilerParams(
            dimension_semantics=("parallel", "parallel")),
    )(x, gathered, distances.reshape(nb, 1, TN), W1[:H], W1[H:],
      b1.reshape(1, H), W2, b2.reshape(1, H))
    return out


# speedup vs baseline: 1.7337x; 1.0362x over previous
import functools

import jax
import jax.numpy as jnp
from jax import lax
from jax.experimental import pallas as pl
from jax.experimental.pallas import tpu as pltpu
from jax.experimental.pallas import tpu_sc as plsc

NC = 2
NS = 16
NW = NC * NS
BS = 128
NBUF = 3
LA = NBUF - 1


def _sc_gather(table, idx3, nblk_per_w, h):
    nb = nblk_per_w
    assert nb >= 8
    mesh = plsc.VectorSubcoreMesh(core_axis_name="c", subcore_axis_name="s")
    total = NW * nb * BS

    @functools.partial(
        pl.kernel,
        out_type=jax.ShapeDtypeStruct((total, h), jnp.float32),
        mesh=mesh,
        scratch_types=[
            pltpu.VMEM((nb, BS), jnp.int32),
            [pltpu.VMEM((BS, h), jnp.float32)] * NBUF,
            [pltpu.SemaphoreType.DMA] * NBUF,
            [pltpu.SemaphoreType.DMA] * NBUF,
        ],
    )
    def k(table_hbm, idx_hbm, out_hbm, idx_v, bufs, gsems, ssems):
        wid = lax.axis_index("s") * NC + lax.axis_index("c")
        pltpu.sync_copy(idx_hbm.at[wid], idx_v)

        def start_gather(j, i):
            pltpu.async_copy(table_hbm.at[idx_v.at[j]], bufs[i], gsems[i])

        def wait_gather(i):
            pltpu.make_async_copy(
                table_hbm.at[idx_v.at[0]], bufs[i], gsems[i]).wait()

        def start_store(j, i):
            pltpu.async_copy(
                bufs[i], out_hbm.at[pl.ds((j * NW + wid) * BS, BS)], ssems[i])

        def wait_store(i):
            pltpu.make_async_copy(
                bufs[i], out_hbm.at[pl.ds(0, BS)], ssems[i]).wait()

        def step_full(j):
            i, i3 = j % NBUF, (j + LA) % NBUF
            wait_gather(i)
            start_store(j, i)
            wait_store(i3)
            start_gather(j + LA, i3)

        def step_nofetch(j):
            wait_gather(j % NBUF)
            start_store(j, j % NBUF)

        E = ((nb - LA) // NBUF) * NBUF
        for j in range(LA):
            start_gather(j, j)
        wait_gather(0)
        start_store(0, 0)
        start_gather(LA, LA)
        for j in range(1, NBUF):
            step_full(j)

        @pl.loop(NBUF, E, step=NBUF)
        def _(j0):
            for i in range(NBUF):
                wait_gather(i)
                start_store(j0 + i, i)
                wait_store((i + LA) % NBUF)
                start_gather(j0 + i + LA, (i + LA) % NBUF)

        for j in range(E, nb - LA):
            step_full(j)
        for j in range(nb - LA, nb):
            step_nofetch(j)
        for i in range(NBUF):
            wait_store(i)

    return k(table, idx3)


def _mlp_body(x_ref, g_ref, d_ref, w1a_ref, w1b_ref, b1_ref, w2_ref, b2_ref,
              o_ref):
    inv = (1.0 / (d_ref[...] + 1e-8)).reshape(-1, 1)
    h = jnp.dot(x_ref[0], w1a_ref[...], preferred_element_type=jnp.float32)
    h = h + inv * jnp.dot(g_ref[...], w1b_ref[...],
                          preferred_element_type=jnp.float32)
    h = jnp.maximum(h + b1_ref[...], 0.0)
    o_ref[0] = (jnp.dot(h, w2_ref[...], preferred_element_type=jnp.float32)
                + b2_ref[...])


def kernel(x, x_scale, fine2coarse_index, distances, W1, b1, W2, b2):
    B, N, H = x.shape
    R = B * N
    nblk = -(-R // BS)
    nblk = -(-nblk // NW) * NW
    Rpad = nblk * BS
    nblk_per_w = nblk // NW

    idx = fine2coarse_index.astype(jnp.int32)
    offs = (jnp.arange(B, dtype=jnp.int32) * N)[:, None]
    idx_all = (idx[None, :] + offs).reshape(R)
    idx_pad = jnp.concatenate([idx_all, jnp.zeros(Rpad - R, jnp.int32)])
    idx3 = idx_pad.reshape(nblk_per_w, NW, BS).transpose(1, 0, 2)
    table = x_scale.reshape(R, H)

    gathered = _sc_gather(table, idx3, nblk_per_w, H)

    TN = 5000
    nb = N // TN
    out = pl.pallas_call(
        _mlp_body,
        grid=(B, nb),
        in_specs=[
            pl.BlockSpec((1, TN, H), lambda b, n: (b, n, 0)),
            pl.BlockSpec((TN, H), lambda b, n: (b * nb + n, 0)),
            pl.BlockSpec((1, 1, TN), lambda b, n: (n, 0, 0)),
            pl.BlockSpec((H, H), lambda b, n: (0, 0)),
            pl.BlockSpec((H, H), lambda b, n: (0, 0)),
            pl.BlockSpec((1, H), lambda b, n: (0, 0)),
            pl.BlockSpec((H, H), lambda b, n: (0, 0)),
            pl.BlockSpec((1, H), lambda b, n: (0, 0)),
        ],
        out_specs=pl.BlockSpec((1, TN, H), lambda b, n: (b, n, 0)),
        out_shape=jax.ShapeDtypeStruct((B, N, H), jnp.float32),
        compiler_params=pltpu.CompilerParams(
            dimension_semantics=("parallel", "parallel")),
    )(x, gathered, distances.reshape(nb, 1, TN), W1[:H], W1[H:],
      b1.reshape(1, H), W2, b2.reshape(1, H))
    return out
